# baseline (speedup 1.0000x reference)
import functools

import jax
import jax.numpy as jnp
from jax import lax
from jax.experimental import pallas as pl
from jax.experimental.pallas import tpu as pltpu
from jax.experimental.pallas import tpu_sc as plsc

B = 16
TOTAL = 32768
D = 512
OUT = 32
NC = 2
NS = 16
NW = NC * NS
L = 16
G = D // L
HALF = G // 2
CH = 16
S0 = 2048
RB = 2048
NBLK = (TOTAL - S0) // RB


def _sc_body(images_hbm, counts_hbm, partial_hbm, counts_v, buf0, buf1, acc,
             sem0, sem1):
    cid = lax.axis_index("c")
    sid = lax.axis_index("s")
    wid = sid * NC + cid

    pltpu.sync_copy(counts_hbm, counts_v)
    counts = counts_v[...]
    run = jnp.int32(0)
    starts_s = []
    ends_s = []
    for b in range(B):
        starts_s.append(run)
        run = run + counts[b]
        ends_s.append(run)
    total = run
    m = jnp.minimum(total, S0)
    per = (m + NW - 1) // NW
    lo = jnp.minimum(wid * per, m)
    hi = jnp.minimum(lo + per, m)

    zero = jnp.zeros((L,), jnp.float32)
    bufs = (buf0, buf1)
    sems = (sem0, sem1)

    def zero_body(bb, carry):
        for g in range(G):
            acc[bb, pl.ds(g * L, L)] = zero
        return carry

    lax.fori_loop(0, B, zero_body, 0)

    a0 = (lo // 8) * 8
    nch = (hi - a0 + CH - 1) // CH

    def chunk_cc(j):
        return jnp.minimum(a0 + j * CH, TOTAL - CH)

    def issue(j, ph):
        pltpu.async_copy(images_hbm.at[pl.ds(chunk_cc(j), CH), :],
                         bufs[ph], sems[ph])

    def wait(j, ph):
        pltpu.make_async_copy(images_hbm.at[pl.ds(chunk_cc(j), CH), :],
                              bufs[ph], sems[ph]).wait()

    @pl.when(nch > 0)
    def _():
        issue(0, 0)

    def consume(j, ph):
        c = a0 + j * CH
        cc = chunk_cc(j)
        buf = bufs[ph]
        for b in range(B):
            o_lo = jnp.maximum(jnp.maximum(lo, starts_s[b]), c) - cc
            o_hi = jnp.minimum(jnp.minimum(hi, ends_s[b]), c + CH) - cc

            @pl.when(o_hi > o_lo)
            def _(b=b, o_lo=o_lo, o_hi=o_hi, buf=buf):
                for half in range(2):
                    def row_body(r, acc_h, half=half, buf=buf):
                        return tuple(
                            acc_h[g] + buf[r, pl.ds((half * HALF + g) * L, L)]
                            for g in range(HALF)
                        )
                    init = tuple(zero for _ in range(HALF))
                    sums = lax.fori_loop(o_lo, o_hi, row_body, init)
                    for g in range(HALF):
                        plsc.addupdate(
                            acc.at[b, pl.ds((half * HALF + g) * L, L)],
                            sums[g])

    def pair_body(jj, carry):
        for ph in range(2):
            j = jj * 2 + ph

            @pl.when(j < nch)
            def _(j=j, ph=ph):
                wait(j, ph)

                @pl.when(j + 1 < nch)
                def _(j=j, ph=ph):
                    issue(j + 1, 1 - ph)

                consume(j, ph)
        return carry

    lax.fori_loop(0, (nch + 1) // 2, pair_body, 0)

    pltpu.sync_copy(acc, partial_hbm.at[wid])


_sc_mesh = plsc.VectorSubcoreMesh(
    core_axis_name="c", subcore_axis_name="s", num_cores=NC, num_subcores=NS
)

_sc_segment_sums = functools.partial(
    pl.kernel,
    out_type=jax.ShapeDtypeStruct((NW, B, D), jnp.float32),
    mesh=_sc_mesh,
    scratch_types=[
        pltpu.VMEM((L,), jnp.int32),
        pltpu.VMEM((CH, D), jnp.float32),
        pltpu.VMEM((CH, D), jnp.float32),
        pltpu.VMEM((B, D), jnp.float32),
        pltpu.SemaphoreType.DMA,
        pltpu.SemaphoreType.DMA,
    ],
)(_sc_body)


def _tcdyn_body(meta_ref, counts_ref, img_ref, out_ref):
    i = pl.program_id(0)
    nb = meta_ref[0]

    @pl.when(i == 0)
    def _():
        out_ref[...] = jnp.zeros((B, D), jnp.float32)

    @pl.when(i < nb)
    def _():
        t0 = S0 + i * RB
        t = lax.broadcasted_iota(jnp.int32, (1, RB), 1) + t0
        run = jnp.int32(0)
        bid = jnp.zeros((1, RB), jnp.int32)
        for b in range(B):
            run = run + counts_ref[b]
            bid = bid + jnp.where(t >= run, 1, 0)
        biota = lax.broadcasted_iota(jnp.int32, (B, RB), 0)
        mask = jnp.where(bid == biota, 1.0, 0.0)
        out_ref[...] += jnp.dot(mask, img_ref[...],
                                preferred_element_type=jnp.float32)


def _tc_dyn_sums(meta, counts, images):
    grid_spec = pltpu.PrefetchScalarGridSpec(
        num_scalar_prefetch=2,
        grid=(NBLK,),
        in_specs=[
            pl.BlockSpec(
                (RB, D),
                lambda i, meta, cnts: (
                    S0 // RB + jnp.minimum(i, jnp.maximum(meta[0] - 1, 0)), 0),
            ),
        ],
        out_specs=pl.BlockSpec((B, D), lambda i, meta, cnts: (0, 0)),
    )
    return pl.pallas_call(
        _tcdyn_body,
        grid_spec=grid_spec,
        out_shape=jax.ShapeDtypeStruct((B, D), jnp.float32),
    )(meta, counts, images)


def _combine_body(partial_ref, segtc_ref, counts_ref, audio_ref, wa_ref,
                  ba_ref, wi_ref, bi_ref, out_ref):
    counts = counts_ref[0, :]
    cf = counts.astype(jnp.float32)
    seg = jnp.sum(partial_ref[...], axis=0) + segtc_ref[...]
    img = jnp.dot(seg, wi_ref[...], preferred_element_type=jnp.float32)
    aud = jnp.dot(audio_ref[...], wa_ref[...], preferred_element_type=jnp.float32)
    res = img + aud + ba_ref[...] + cf[:, None] * bi_ref[...]
    out_ref[...] = res * (1.0 / (1.0 + cf))[:, None]


def kernel(audio, images, counts, W_audio, b_audio, W_image, b_image):
    partial = _sc_segment_sums(images, counts)
    total = jnp.sum(counts)
    rem = jnp.maximum(total - S0, 0)
    meta = ((rem + RB - 1) // RB).reshape(1).astype(jnp.int32)
    seg_tc = _tc_dyn_sums(meta, counts, images)
    return pl.pallas_call(
        _combine_body,
        out_shape=jax.ShapeDtypeStruct((B, OUT), jnp.float32),
    )(
        partial,
        seg_tc,
        counts.reshape(1, B),
        audio,
        W_audio,
        b_audio.reshape(1, OUT),
        W_image,
        b_image.reshape(1, OUT),
    )


# speedup vs baseline: 1.0676x; 1.0676x over previous
import functools

import jax
import jax.numpy as jnp
from jax import lax
from jax.experimental import pallas as pl
from jax.experimental.pallas import tpu as pltpu
from jax.experimental.pallas import tpu_sc as plsc

B = 16
TOTAL = 32768
D = 512
OUT = 32
NC = 2
NS = 16
NW = NC * NS
L = 16
G = D // L
HALF = G // 2
CH = 32
S0 = 2048
RB = 2048
NBLK = (TOTAL - S0) // RB


def _sc_body(images_hbm, counts_hbm, partial_hbm, counts_v, buf0, buf1, acc,
             sem0, sem1):
    cid = lax.axis_index("c")
    sid = lax.axis_index("s")
    wid = sid * NC + cid

    pltpu.sync_copy(counts_hbm, counts_v)
    counts = counts_v[...]
    run = jnp.int32(0)
    starts_s = []
    ends_s = []
    for b in range(B):
        starts_s.append(run)
        run = run + counts[b]
        ends_s.append(run)
    total = run
    m = jnp.minimum(total, S0)
    per = (m + NW - 1) // NW
    lo = jnp.minimum(wid * per, m)
    hi = jnp.minimum(lo + per, m)

    zero = jnp.zeros((L,), jnp.float32)
    bufs = (buf0, buf1)
    sems = (sem0, sem1)

    def zero_body(bb, carry):
        for g in range(G):
            acc[bb, pl.ds(g * L, L)] = zero
        return carry

    lax.fori_loop(0, B, zero_body, 0)

    a0 = (lo // 8) * 8
    nch = (hi - a0 + CH - 1) // CH

    def chunk_cc(j):
        return jnp.minimum(a0 + j * CH, TOTAL - CH)

    def issue(j, ph):
        pltpu.async_copy(images_hbm.at[pl.ds(chunk_cc(j), CH), :],
                         bufs[ph], sems[ph])

    def wait(j, ph):
        pltpu.make_async_copy(images_hbm.at[pl.ds(chunk_cc(j), CH), :],
                              bufs[ph], sems[ph]).wait()

    @pl.when(nch > 0)
    def _():
        issue(0, 0)

    def consume(j, ph):
        c = a0 + j * CH
        cc = chunk_cc(j)
        buf = bufs[ph]
        for b in range(B):
            o_lo = jnp.maximum(jnp.maximum(lo, starts_s[b]), c) - cc
            o_hi = jnp.minimum(jnp.minimum(hi, ends_s[b]), c + CH) - cc

            @pl.when(o_hi > o_lo)
            def _(b=b, o_lo=o_lo, o_hi=o_hi, buf=buf):
                for half in range(2):
                    def row_body(r, acc_h, half=half, buf=buf):
                        return tuple(
                            acc_h[g] + buf[r, pl.ds((half * HALF + g) * L, L)]
                            for g in range(HALF)
                        )
                    init = tuple(zero for _ in range(HALF))
                    sums = lax.fori_loop(o_lo, o_hi, row_body, init)
                    for g in range(HALF):
                        plsc.addupdate(
                            acc.at[b, pl.ds((half * HALF + g) * L, L)],
                            sums[g])

    def pair_body(jj, carry):
        for ph in range(2):
            j = jj * 2 + ph

            @pl.when(j < nch)
            def _(j=j, ph=ph):
                wait(j, ph)

                @pl.when(j + 1 < nch)
                def _(j=j, ph=ph):
                    issue(j + 1, 1 - ph)

                consume(j, ph)
        return carry

    lax.fori_loop(0, (nch + 1) // 2, pair_body, 0)

    pltpu.sync_copy(acc, partial_hbm.at[wid])


_sc_mesh = plsc.VectorSubcoreMesh(
    core_axis_name="c", subcore_axis_name="s", num_cores=NC, num_subcores=NS
)

_sc_segment_sums = functools.partial(
    pl.kernel,
    out_type=jax.ShapeDtypeStruct((NW, B, D), jnp.float32),
    mesh=_sc_mesh,
    scratch_types=[
        pltpu.VMEM((L,), jnp.int32),
        pltpu.VMEM((CH, D), jnp.float32),
        pltpu.VMEM((CH, D), jnp.float32),
        pltpu.VMEM((B, D), jnp.float32),
        pltpu.SemaphoreType.DMA,
        pltpu.SemaphoreType.DMA,
    ],
)(_sc_body)


def _tcdyn_body(meta_ref, counts_ref, img_ref, out_ref):
    i = pl.program_id(0)
    nb = meta_ref[0]

    @pl.when(i == 0)
    def _():
        out_ref[...] = jnp.zeros((B, D), jnp.float32)

    @pl.when(i < nb)
    def _():
        t0 = S0 + i * RB
        t = lax.broadcasted_iota(jnp.int32, (1, RB), 1) + t0
        run = jnp.int32(0)
        bid = jnp.zeros((1, RB), jnp.int32)
        for b in range(B):
            run = run + counts_ref[b]
            bid = bid + jnp.where(t >= run, 1, 0)
        biota = lax.broadcasted_iota(jnp.int32, (B, RB), 0)
        mask = jnp.where(bid == biota, 1.0, 0.0)
        out_ref[...] += jnp.dot(mask, img_ref[...],
                                preferred_element_type=jnp.float32)


def _tc_dyn_sums(meta, counts, images):
    grid_spec = pltpu.PrefetchScalarGridSpec(
        num_scalar_prefetch=2,
        grid=(NBLK,),
        in_specs=[
            pl.BlockSpec(
                (RB, D),
                lambda i, meta, cnts: (
                    S0 // RB + jnp.minimum(i, jnp.maximum(meta[0] - 1, 0)), 0),
            ),
        ],
        out_specs=pl.BlockSpec((B, D), lambda i, meta, cnts: (0, 0)),
    )
    return pl.pallas_call(
        _tcdyn_body,
        grid_spec=grid_spec,
        out_shape=jax.ShapeDtypeStruct((B, D), jnp.float32),
    )(meta, counts, images)


def _combine_body(partial_ref, segtc_ref, counts_ref, audio_ref, wa_ref,
                  ba_ref, wi_ref, bi_ref, out_ref):
    counts = counts_ref[0, :]
    cf = counts.astype(jnp.float32)
    seg = jnp.sum(partial_ref[...], axis=0) + segtc_ref[...]
    img = jnp.dot(seg, wi_ref[...], preferred_element_type=jnp.float32)
    aud = jnp.dot(audio_ref[...], wa_ref[...], preferred_element_type=jnp.float32)
    res = img + aud + ba_ref[...] + cf[:, None] * bi_ref[...]
    out_ref[...] = res * (1.0 / (1.0 + cf))[:, None]


def kernel(audio, images, counts, W_audio, b_audio, W_image, b_image):
    partial = _sc_segment_sums(images, counts)
    total = jnp.sum(counts)
    rem = jnp.maximum(total - S0, 0)
    meta = ((rem + RB - 1) // RB).reshape(1).astype(jnp.int32)
    seg_tc = _tc_dyn_sums(meta, counts, images)
    return pl.pallas_call(
        _combine_body,
        out_shape=jax.ShapeDtypeStruct((B, OUT), jnp.float32),
    )(
        partial,
        seg_tc,
        counts.reshape(1, B),
        audio,
        W_audio,
        b_audio.reshape(1, OUT),
        W_image,
        b_image.reshape(1, OUT),
    )


# speedup vs baseline: 1.1036x; 1.0337x over previous
import functools

import jax
import jax.numpy as jnp
from jax import lax
from jax.experimental import pallas as pl
from jax.experimental.pallas import tpu as pltpu
from jax.experimental.pallas import tpu_sc as plsc

B = 16
TOTAL = 32768
D = 512
OUT = 32
NC = 2
NS = 16
NW = NC * NS
L = 16
G = D // L
HALF = G // 2
CH = 32
S0 = 512
RB = 2048
FB = S0 // RB
NBLK = TOTAL // RB - FB


def _sc_body(images_hbm, counts_hbm, partial_hbm, counts_v, buf0, buf1, acc,
             sem0, sem1):
    cid = lax.axis_index("c")
    sid = lax.axis_index("s")
    wid = sid * NC + cid

    pltpu.sync_copy(counts_hbm, counts_v)
    counts = counts_v[...]
    run = jnp.int32(0)
    starts_s = []
    ends_s = []
    for b in range(B):
        starts_s.append(run)
        run = run + counts[b]
        ends_s.append(run)
    total = run
    m = jnp.minimum(total, S0)
    per = (m + NW - 1) // NW
    lo = jnp.minimum(wid * per, m)
    hi = jnp.minimum(lo + per, m)

    zero = jnp.zeros((L,), jnp.float32)
    bufs = (buf0, buf1)
    sems = (sem0, sem1)

    def zero_body(bb, carry):
        for g in range(G):
            acc[bb, pl.ds(g * L, L)] = zero
        return carry

    lax.fori_loop(0, B, zero_body, 0)

    a0 = (lo // 8) * 8
    nch = (hi - a0 + CH - 1) // CH

    def chunk_cc(j):
        return jnp.minimum(a0 + j * CH, TOTAL - CH)

    def issue(j, ph):
        pltpu.async_copy(images_hbm.at[pl.ds(chunk_cc(j), CH), :],
                         bufs[ph], sems[ph])

    def wait(j, ph):
        pltpu.make_async_copy(images_hbm.at[pl.ds(chunk_cc(j), CH), :],
                              bufs[ph], sems[ph]).wait()

    @pl.when(nch > 0)
    def _():
        issue(0, 0)

    def consume(j, ph):
        c = a0 + j * CH
        cc = chunk_cc(j)
        buf = bufs[ph]
        for b in range(B):
            o_lo = jnp.maximum(jnp.maximum(lo, starts_s[b]), c) - cc
            o_hi = jnp.minimum(jnp.minimum(hi, ends_s[b]), c + CH) - cc

            @pl.when(o_hi > o_lo)
            def _(b=b, o_lo=o_lo, o_hi=o_hi, buf=buf):
                for half in range(2):
                    def row_body(r, acc_h, half=half, buf=buf):
                        return tuple(
                            acc_h[g] + buf[r, pl.ds((half * HALF + g) * L, L)]
                            for g in range(HALF)
                        )
                    init = tuple(zero for _ in range(HALF))
                    sums = lax.fori_loop(o_lo, o_hi, row_body, init)
                    for g in range(HALF):
                        plsc.addupdate(
                            acc.at[b, pl.ds((half * HALF + g) * L, L)],
                            sums[g])

    def pair_body(jj, carry):
        for ph in range(2):
            j = jj * 2 + ph

            @pl.when(j < nch)
            def _(j=j, ph=ph):
                wait(j, ph)

                @pl.when(j + 1 < nch)
                def _(j=j, ph=ph):
                    issue(j + 1, 1 - ph)

                consume(j, ph)
        return carry

    lax.fori_loop(0, (nch + 1) // 2, pair_body, 0)

    pltpu.sync_copy(acc, partial_hbm.at[wid])


_sc_mesh = plsc.VectorSubcoreMesh(
    core_axis_name="c", subcore_axis_name="s", num_cores=NC, num_subcores=NS
)

_sc_segment_sums = functools.partial(
    pl.kernel,
    out_type=jax.ShapeDtypeStruct((NW, B, D), jnp.float32),
    mesh=_sc_mesh,
    scratch_types=[
        pltpu.VMEM((L,), jnp.int32),
        pltpu.VMEM((CH, D), jnp.float32),
        pltpu.VMEM((CH, D), jnp.float32),
        pltpu.VMEM((B, D), jnp.float32),
        pltpu.SemaphoreType.DMA,
        pltpu.SemaphoreType.DMA,
    ],
)(_sc_body)


def _tcdyn_body(meta_ref, counts_ref, img_ref, out_ref):
    i = pl.program_id(0)
    nb = meta_ref[0]

    @pl.when(i == 0)
    def _():
        out_ref[...] = jnp.zeros((B, D), jnp.float32)

    @pl.when(i < nb)
    def _():
        t0 = (FB + i) * RB
        t = lax.broadcasted_iota(jnp.int32, (1, RB), 1) + t0
        run = jnp.int32(0)
        bid = jnp.zeros((1, RB), jnp.int32)
        for b in range(B):
            run = run + counts_ref[b]
            bid = bid + jnp.where(t >= run, 1, 0)
        biota = lax.broadcasted_iota(jnp.int32, (B, RB), 0)
        mask = jnp.where((bid == biota) & (t >= S0), 1.0, 0.0)
        out_ref[...] += jnp.dot(mask, img_ref[...],
                                preferred_element_type=jnp.float32)


def _tc_dyn_sums(meta, counts, images):
    grid_spec = pltpu.PrefetchScalarGridSpec(
        num_scalar_prefetch=2,
        grid=(NBLK,),
        in_specs=[
            pl.BlockSpec(
                (RB, D),
                lambda i, meta, cnts: (
                    FB + jnp.minimum(i, jnp.maximum(meta[0] - 1, 0)), 0),
            ),
        ],
        out_specs=pl.BlockSpec((B, D), lambda i, meta, cnts: (0, 0)),
    )
    return pl.pallas_call(
        _tcdyn_body,
        grid_spec=grid_spec,
        out_shape=jax.ShapeDtypeStruct((B, D), jnp.float32),
    )(meta, counts, images)


def _combine_body(partial_ref, segtc_ref, counts_ref, audio_ref, wa_ref,
                  ba_ref, wi_ref, bi_ref, out_ref):
    counts = counts_ref[0, :]
    cf = counts.astype(jnp.float32)
    seg = jnp.sum(partial_ref[...], axis=0) + segtc_ref[...]
    img = jnp.dot(seg, wi_ref[...], preferred_element_type=jnp.float32)
    aud = jnp.dot(audio_ref[...], wa_ref[...], preferred_element_type=jnp.float32)
    res = img + aud + ba_ref[...] + cf[:, None] * bi_ref[...]
    out_ref[...] = res * (1.0 / (1.0 + cf))[:, None]


def kernel(audio, images, counts, W_audio, b_audio, W_image, b_image):
    partial = _sc_segment_sums(images, counts)
    total = jnp.sum(counts)
    nb = jnp.where(total > S0, (total + RB - 1) // RB - FB, 0)
    meta = nb.reshape(1).astype(jnp.int32)
    seg_tc = _tc_dyn_sums(meta, counts, images)
    return pl.pallas_call(
        _combine_body,
        out_shape=jax.ShapeDtypeStruct((B, OUT), jnp.float32),
    )(
        partial,
        seg_tc,
        counts.reshape(1, B),
        audio,
        W_audio,
        b_audio.reshape(1, OUT),
        W_image,
        b_image.reshape(1, OUT),
    )


# speedup vs baseline: 1.1071x; 1.0032x over previous
import functools

import jax
import jax.numpy as jnp
from jax import lax
from jax.experimental import pallas as pl
from jax.experimental.pallas import tpu as pltpu
from jax.experimental.pallas import tpu_sc as plsc

B = 16
TOTAL = 32768
D = 512
OUT = 32
NC = 2
NS = 16
NW = NC * NS
L = 16
G = D // L
HALF = G // 2
CH = 32
S0 = 256
RB = 2048
FB = S0 // RB
NBLK = TOTAL // RB - FB


def _sc_body(images_hbm, counts_hbm, partial_hbm, counts_v, buf0, buf1, acc,
             sem0, sem1, semc):
    cid = lax.axis_index("c")
    sid = lax.axis_index("s")
    wid = sid * NC + cid

    pltpu.async_copy(counts_hbm, counts_v, semc)

    zero = jnp.zeros((L,), jnp.float32)
    bufs = (buf0, buf1)
    sems = (sem0, sem1)

    def zero_body(bb, carry):
        for g in range(G):
            acc[bb, pl.ds(g * L, L)] = zero
        return carry

    lax.fori_loop(0, B, zero_body, 0)

    pltpu.make_async_copy(counts_hbm, counts_v, semc).wait()
    counts = counts_v[...]
    run = jnp.int32(0)
    starts_s = []
    ends_s = []
    for b in range(B):
        starts_s.append(run)
        run = run + counts[b]
        ends_s.append(run)
    total = run
    m = jnp.minimum(total, S0)
    per = (m + NW - 1) // NW
    lo = jnp.minimum(wid * per, m)
    hi = jnp.minimum(lo + per, m)

    a0 = (lo // 8) * 8
    nch = (hi - a0 + CH - 1) // CH

    def chunk_cc(j):
        return jnp.minimum(a0 + j * CH, TOTAL - CH)

    def issue(j, ph):
        pltpu.async_copy(images_hbm.at[pl.ds(chunk_cc(j), CH), :],
                         bufs[ph], sems[ph])

    def wait(j, ph):
        pltpu.make_async_copy(images_hbm.at[pl.ds(chunk_cc(j), CH), :],
                              bufs[ph], sems[ph]).wait()

    @pl.when(nch > 0)
    def _():
        issue(0, 0)

    def consume(j, ph):
        c = a0 + j * CH
        cc = chunk_cc(j)
        buf = bufs[ph]
        for b in range(B):
            o_lo = jnp.maximum(jnp.maximum(lo, starts_s[b]), c) - cc
            o_hi = jnp.minimum(jnp.minimum(hi, ends_s[b]), c + CH) - cc

            @pl.when(o_hi > o_lo)
            def _(b=b, o_lo=o_lo, o_hi=o_hi, buf=buf):
                for half in range(2):
                    def row_body(r, acc_h, half=half, buf=buf):
                        return tuple(
                            acc_h[g] + buf[r, pl.ds((half * HALF + g) * L, L)]
                            for g in range(HALF)
                        )
                    init = tuple(zero for _ in range(HALF))
                    sums = lax.fori_loop(o_lo, o_hi, row_body, init)
                    for g in range(HALF):
                        plsc.addupdate(
                            acc.at[b, pl.ds((half * HALF + g) * L, L)],
                            sums[g])

    def pair_body(jj, carry):
        for ph in range(2):
            j = jj * 2 + ph

            @pl.when(j < nch)
            def _(j=j, ph=ph):
                wait(j, ph)

                @pl.when(j + 1 < nch)
                def _(j=j, ph=ph):
                    issue(j + 1, 1 - ph)

                consume(j, ph)
        return carry

    lax.fori_loop(0, (nch + 1) // 2, pair_body, 0)

    pltpu.sync_copy(acc, partial_hbm.at[wid])


_sc_mesh = plsc.VectorSubcoreMesh(
    core_axis_name="c", subcore_axis_name="s", num_cores=NC, num_subcores=NS
)

_sc_segment_sums = functools.partial(
    pl.kernel,
    out_type=jax.ShapeDtypeStruct((NW, B, D), jnp.float32),
    mesh=_sc_mesh,
    scratch_types=[
        pltpu.VMEM((L,), jnp.int32),
        pltpu.VMEM((CH, D), jnp.float32),
        pltpu.VMEM((CH, D), jnp.float32),
        pltpu.VMEM((B, D), jnp.float32),
        pltpu.SemaphoreType.DMA,
        pltpu.SemaphoreType.DMA,
        pltpu.SemaphoreType.DMA,
    ],
)(_sc_body)


def _tcdyn_body(meta_ref, counts_ref, img_ref, out_ref):
    i = pl.program_id(0)
    nb = meta_ref[0]

    @pl.when(i == 0)
    def _():
        out_ref[...] = jnp.zeros((B, D), jnp.float32)

    @pl.when(i < nb)
    def _():
        t0 = (FB + i) * RB
        t = lax.broadcasted_iota(jnp.int32, (1, RB), 1) + t0
        run = jnp.int32(0)
        bid = jnp.zeros((1, RB), jnp.int32)
        for b in range(B):
            run = run + counts_ref[b]
            bid = bid + jnp.where(t >= run, 1, 0)
        biota = lax.broadcasted_iota(jnp.int32, (B, RB), 0)
        mask = jnp.where((bid == biota) & (t >= S0), 1.0, 0.0)
        out_ref[...] += jnp.dot(mask, img_ref[...],
                                preferred_element_type=jnp.float32)


def _tc_dyn_sums(meta, counts, images):
    grid_spec = pltpu.PrefetchScalarGridSpec(
        num_scalar_prefetch=2,
        grid=(NBLK,),
        in_specs=[
            pl.BlockSpec(
                (RB, D),
                lambda i, meta, cnts: (
                    FB + jnp.minimum(i, jnp.maximum(meta[0] - 1, 0)), 0),
            ),
        ],
        out_specs=pl.BlockSpec((B, D), lambda i, meta, cnts: (0, 0)),
    )
    return pl.pallas_call(
        _tcdyn_body,
        grid_spec=grid_spec,
        out_shape=jax.ShapeDtypeStruct((B, D), jnp.float32),
    )(meta, counts, images)


def _combine_body(partial_ref, segtc_ref, counts_ref, audio_ref, wa_ref,
                  ba_ref, wi_ref, bi_ref, out_ref):
    cf = counts_ref[...].astype(jnp.float32)
    seg = jnp.sum(partial_ref[...], axis=0) + segtc_ref[...]
    img = jnp.dot(seg, wi_ref[...], preferred_element_type=jnp.float32)
    aud = jnp.dot(audio_ref[...], wa_ref[...], preferred_element_type=jnp.float32)
    res = img + aud + ba_ref[...][None, :] + cf[:, None] * bi_ref[...][None, :]
    out_ref[...] = res * (1.0 / (1.0 + cf))[:, None]


def kernel(audio, images, counts, W_audio, b_audio, W_image, b_image):
    partial = _sc_segment_sums(images, counts)
    total = jnp.sum(counts)
    nb = jnp.where(total > S0, (total + RB - 1) // RB - FB, 0)
    meta = nb.reshape(1).astype(jnp.int32)
    seg_tc = _tc_dyn_sums(meta, counts, images)
    return pl.pallas_call(
        _combine_body,
        out_shape=jax.ShapeDtypeStruct((B, OUT), jnp.float32),
    )(
        partial,
        seg_tc,
        counts,
        audio,
        W_audio,
        b_audio,
        W_image,
        b_image,
    )
